# baseline (speedup 1.0000x reference)
import functools
import math

import numpy as np
import jax
import jax.numpy as jnp
from jax.experimental import pallas as pl
from jax.experimental.pallas import tpu as pltpu


def _build_constants(H, W):
    idx_h = np.arange(H)
    idx_w = np.arange(W)
    fh = np.exp(-2j * np.pi * np.outer(idx_h, idx_h) / H)
    fw = np.exp(-2j * np.pi * np.outer(idx_w, idx_w) / W)
    f2 = np.kron(fh, fw)
    f2_cat = np.concatenate(
        [np.real(f2), np.imag(f2)], axis=1).astype(np.float32)

    HW = H * W
    cmask = np.zeros((3, HW), np.float32)
    rmask = np.zeros((3, HW), np.float32)
    for i, d in enumerate((-1, 0, 1)):
        mc = np.zeros((H, W), np.float32)
        mc[:, max(0, -d):min(W, W - d)] = 1.0
        cmask[i] = mc.reshape(-1)
        mr = np.zeros((H, W), np.float32)
        mr[max(0, -d):min(H, H - d), :] = 1.0
        rmask[i] = mr.reshape(-1)
    cmask2 = np.concatenate([cmask, cmask], axis=1)
    rmask2 = np.concatenate([rmask, rmask], axis=1)
    return f2_cat, cmask2, rmask2


def _bf16_dot(a, b):
    return jax.lax.dot_general(
        a, b, (((1,), (0,)), ((), ())),
        preferred_element_type=jnp.float32)


def _fused_kernel(x_ref, f2cat_ref, f2stk_ref, cmask_ref, rmask_ref, w_ref,
                  b_ref, y_ref, e_ref, *, W, inv_hw, G, Cin, Cout):
    HW = f2cat_ref.shape[0]
    n2 = 2 * HW
    x = x_ref[...].reshape(G * Cin, HW).astype(jnp.bfloat16)
    yri_all = _bf16_dot(x, f2cat_ref[...]).astype(jnp.bfloat16)
    cmask = cmask_ref[...]
    rmask = rmask_ref[...]

    ustk_blocks = []
    for g in range(G):
        yri = yri_all[g * Cin:(g + 1) * Cin]
        us = []
        for j, dw in enumerate((-1, 0, 1)):
            sm = dw % n2
            if sm == 0:
                us.append(yri)
            else:
                u = jnp.concatenate([yri[:, sm:], yri[:, :sm]], axis=1)
                us.append(u * cmask[j:j + 1, :])
        ustk_blocks.append(jnp.concatenate(us, axis=0))
    ustk = jnp.concatenate(ustk_blocks, axis=1)
    p_all = _bf16_dot(w_ref[...], ustk)

    c_blocks = []
    for g in range(G):
        p = p_all[:, g * n2:(g + 1) * n2]
        acc = None
        for i, dh in enumerate((-1, 0, 1)):
            pi = p[i * Cout:(i + 1) * Cout]
            sm = (dh * W) % n2
            if sm == 0:
                term = pi
            else:
                term = jnp.concatenate([pi[:, sm:], pi[:, :sm]], axis=1)
                term = term * rmask[i:i + 1, :]
            acc = term if acc is None else acc + term
        c = acc + b_ref[...]

        sq = c * c
        e_ref[g] = jnp.sum(jnp.sum(sq, axis=1, keepdims=True),
                           axis=0, keepdims=True)
        c_blocks.append(c.astype(jnp.bfloat16))

    c_all = jnp.concatenate(c_blocks, axis=0)
    y = _bf16_dot(c_all, f2stk_ref[...]) * inv_hw
    y_ref[...] = y.reshape(G, Cout, HW).astype(jnp.bfloat16)


def _scale_kernel(e_ref, y_ref, o_ref, *, inv_n):
    total = jnp.sum(e_ref[...])
    scale = jax.lax.rsqrt(total * inv_n + 1e-8)
    o_ref[...] = y_ref[...].astype(jnp.float32) * scale


def kernel(x, weight, bias):
    B, Cin, H, W = x.shape
    Cout = weight.shape[0]
    HW = H * W
    G = next(g for g in (4, 2, 1) if B % g == 0)

    f2_cat_np, cmask_np, rmask_np = _build_constants(H, W)
    f2_cat = jnp.asarray(f2_cat_np, dtype=jnp.bfloat16)
    f2_stk = jnp.asarray(f2_cat_np.T.copy(), dtype=jnp.bfloat16)
    cmask = jnp.asarray(cmask_np, dtype=jnp.bfloat16)
    rmask = jnp.asarray(rmask_np, dtype=jnp.float32)

    x_flat = x.reshape(B, Cin, HW)
    w_sep = jnp.transpose(weight, (2, 0, 3, 1)).reshape(3 * Cout, 3 * Cin)
    w_sep = w_sep.astype(jnp.bfloat16)
    b_col = bias.reshape(Cout, 1).astype(jnp.float32)

    fused = functools.partial(
        _fused_kernel, W=W, inv_hw=1.0 / float(HW),
        G=G, Cin=Cin, Cout=Cout)
    y_unscaled, esum = pl.pallas_call(
        fused,
        out_shape=(jax.ShapeDtypeStruct((B, Cout, HW), jnp.bfloat16),
                   jax.ShapeDtypeStruct((B, 1, 1), jnp.float32)),
        grid=(B // G,),
        in_specs=[
            pl.BlockSpec((G, Cin, HW), lambda b: (b, 0, 0)),
            pl.BlockSpec((HW, 2 * HW), lambda b: (0, 0)),
            pl.BlockSpec((2 * HW, HW), lambda b: (0, 0)),
            pl.BlockSpec((3, 2 * HW), lambda b: (0, 0)),
            pl.BlockSpec((3, 2 * HW), lambda b: (0, 0)),
            pl.BlockSpec((3 * Cout, 3 * Cin), lambda b: (0, 0)),
            pl.BlockSpec((Cout, 1), lambda b: (0, 0)),
        ],
        out_specs=(
            pl.BlockSpec((G, Cout, HW), lambda b: (b, 0, 0)),
            pl.BlockSpec((G, 1, 1), lambda b: (b, 0, 0)),
        ),
        compiler_params=pltpu.CompilerParams(
            dimension_semantics=("parallel",)),
    )(x_flat, f2_cat, f2_stk, cmask, rmask, w_sep, b_col)

    inv_n = 1.0 / float(B * Cout * HW)
    out = pl.pallas_call(
        functools.partial(_scale_kernel, inv_n=inv_n),
        out_shape=jax.ShapeDtypeStruct((B, Cout, HW), jnp.float32),
        grid=(B // G,),
        in_specs=[
            pl.BlockSpec((B, 1, 1), lambda b: (0, 0, 0)),
            pl.BlockSpec((G, Cout, HW), lambda b: (b, 0, 0)),
        ],
        out_specs=pl.BlockSpec((G, Cout, HW), lambda b: (b, 0, 0)),
        compiler_params=pltpu.CompilerParams(
            dimension_semantics=("parallel",)),
    )(esum, y_unscaled)

    return out.reshape(B, Cout, H, W)


# speedup vs baseline: 1.1132x; 1.1132x over previous
import functools
import math

import numpy as np
import jax
import jax.numpy as jnp
from jax.experimental import pallas as pl
from jax.experimental.pallas import tpu as pltpu


def _build_constants(H, W):
    idx_h = np.arange(H)
    idx_w = np.arange(W)
    fh = np.exp(-2j * np.pi * np.outer(idx_h, idx_h) / H)
    fw = np.exp(-2j * np.pi * np.outer(idx_w, idx_w) / W)
    f2 = np.kron(fh, fw)
    f2_cat = np.concatenate(
        [np.real(f2), np.imag(f2)], axis=1).astype(np.float32)

    HW = H * W
    cmask = np.zeros((3, HW), np.float32)
    rmask = np.zeros((3, HW), np.float32)
    for i, d in enumerate((-1, 0, 1)):
        mc = np.zeros((H, W), np.float32)
        mc[:, max(0, -d):min(W, W - d)] = 1.0
        cmask[i] = mc.reshape(-1)
        mr = np.zeros((H, W), np.float32)
        mr[max(0, -d):min(H, H - d), :] = 1.0
        rmask[i] = mr.reshape(-1)
    cmask2 = np.concatenate([cmask, cmask], axis=1)
    rmask2 = np.concatenate([rmask, rmask], axis=1)
    return f2_cat, cmask2, rmask2


def _bf16_dot(a, b):
    return jax.lax.dot_general(
        a, b, (((1,), (0,)), ((), ())),
        preferred_element_type=jnp.float32)


def _fused_kernel(x_ref, f2cat_ref, f2stk_ref, cmask_ref, rmask_ref, w_ref,
                  b_ref, y_ref, e_ref, *, W, inv_hw, G, Cin, Cout):
    HW = f2cat_ref.shape[0]
    n2 = 2 * HW
    x = x_ref[...].reshape(G * Cin, HW).astype(jnp.bfloat16)
    yri_all = _bf16_dot(x, f2cat_ref[...]).astype(jnp.bfloat16)
    cmask = cmask_ref[...]
    rmask = rmask_ref[...]

    ustk_blocks = []
    for g in range(G):
        yri = yri_all[g * Cin:(g + 1) * Cin]
        us = []
        for j, dw in enumerate((-1, 0, 1)):
            sm = dw % n2
            if sm == 0:
                us.append(yri)
            else:
                u = jnp.concatenate([yri[:, sm:], yri[:, :sm]], axis=1)
                us.append(u * cmask[j:j + 1, :])
        ustk_blocks.append(jnp.concatenate(us, axis=0))
    ustk = jnp.concatenate(ustk_blocks, axis=1)
    p_all = _bf16_dot(w_ref[...], ustk)

    c_blocks = []
    for g in range(G):
        p = p_all[:, g * n2:(g + 1) * n2]
        acc = None
        for i, dh in enumerate((-1, 0, 1)):
            pi = p[i * Cout:(i + 1) * Cout]
            sm = (dh * W) % n2
            if sm == 0:
                term = pi
            else:
                term = jnp.concatenate([pi[:, sm:], pi[:, :sm]], axis=1)
                term = term * rmask[i:i + 1, :]
            acc = term if acc is None else acc + term
        c = acc + b_ref[...]

        sq = c * c
        e_ref[g] = jnp.sum(jnp.sum(sq, axis=1, keepdims=True),
                           axis=0, keepdims=True)
        c_blocks.append(c.astype(jnp.bfloat16))

    c_all = jnp.concatenate(c_blocks, axis=0)
    y = _bf16_dot(c_all, f2stk_ref[...]) * inv_hw
    y_ref[...] = y.reshape(G, Cout, HW).astype(jnp.bfloat16)


def _scale_kernel(e_ref, y_ref, o_ref, *, inv_n):
    total = jnp.sum(e_ref[...])
    scale = jax.lax.rsqrt(total * inv_n + 1e-8)
    o_ref[...] = y_ref[...].astype(jnp.float32) * scale


def kernel(x, weight, bias):
    B, Cin, H, W = x.shape
    Cout = weight.shape[0]
    HW = H * W
    G = next(g for g in (8, 4, 2, 1) if B % g == 0)

    f2_cat_np, cmask_np, rmask_np = _build_constants(H, W)
    f2_cat = jnp.asarray(f2_cat_np, dtype=jnp.bfloat16)
    f2_stk = jnp.asarray(f2_cat_np.T.copy(), dtype=jnp.bfloat16)
    cmask = jnp.asarray(cmask_np, dtype=jnp.bfloat16)
    rmask = jnp.asarray(rmask_np, dtype=jnp.float32)

    x_flat = x.reshape(B, Cin, HW)
    w_sep = jnp.transpose(weight, (2, 0, 3, 1)).reshape(3 * Cout, 3 * Cin)
    w_sep = w_sep.astype(jnp.bfloat16)
    b_col = bias.reshape(Cout, 1).astype(jnp.float32)

    fused = functools.partial(
        _fused_kernel, W=W, inv_hw=1.0 / float(HW),
        G=G, Cin=Cin, Cout=Cout)
    y_unscaled, esum = pl.pallas_call(
        fused,
        out_shape=(jax.ShapeDtypeStruct((B, Cout, HW), jnp.bfloat16),
                   jax.ShapeDtypeStruct((B, 1, 1), jnp.float32)),
        grid=(B // G,),
        in_specs=[
            pl.BlockSpec((G, Cin, HW), lambda b: (b, 0, 0)),
            pl.BlockSpec((HW, 2 * HW), lambda b: (0, 0)),
            pl.BlockSpec((2 * HW, HW), lambda b: (0, 0)),
            pl.BlockSpec((3, 2 * HW), lambda b: (0, 0)),
            pl.BlockSpec((3, 2 * HW), lambda b: (0, 0)),
            pl.BlockSpec((3 * Cout, 3 * Cin), lambda b: (0, 0)),
            pl.BlockSpec((Cout, 1), lambda b: (0, 0)),
        ],
        out_specs=(
            pl.BlockSpec((G, Cout, HW), lambda b: (b, 0, 0)),
            pl.BlockSpec((G, 1, 1), lambda b: (b, 0, 0)),
        ),
        compiler_params=pltpu.CompilerParams(
            dimension_semantics=("parallel",)),
    )(x_flat, f2_cat, f2_stk, cmask, rmask, w_sep, b_col)

    inv_n = 1.0 / float(B * Cout * HW)
    out = pl.pallas_call(
        functools.partial(_scale_kernel, inv_n=inv_n),
        out_shape=jax.ShapeDtypeStruct((B, Cout, HW), jnp.float32),
        grid=(B // G,),
        in_specs=[
            pl.BlockSpec((B, 1, 1), lambda b: (0, 0, 0)),
            pl.BlockSpec((G, Cout, HW), lambda b: (b, 0, 0)),
        ],
        out_specs=pl.BlockSpec((G, Cout, HW), lambda b: (b, 0, 0)),
        compiler_params=pltpu.CompilerParams(
            dimension_semantics=("parallel",)),
    )(esum, y_unscaled)

    return out.reshape(B, Cout, H, W)
